# Initial kernel scaffold; baseline (speedup 1.0000x reference)
import jax
import jax.numpy as jnp
from jax.experimental import pallas as pl


def kernel(cube, grid_mask):
    raise NotImplementedError("write your pallas kernel here")


# speedup vs baseline: 1.5683x; 1.5683x over previous
import functools

import numpy as np
import jax
import jax.numpy as jnp
from jax.experimental import pallas as pl
from jax.experimental.pallas import tpu as pltpu

NPIX = 1024
NCHAN = 8
KF = NPIX // 2 + 1
KPAD = 640
NSEL = 400000
_ARCSEC = np.pi / (180.0 * 3600.0)
_SCALE = np.float32((0.005 * _ARCSEC) ** 2 / _ARCSEC ** 2)

_j = np.arange(NPIX)
_ang_e = (2.0 * np.pi / NPIX) * np.outer(_j, np.arange(KPAD))
_EC = np.cos(_ang_e).astype(np.float32)
_ES = np.sin(_ang_e).astype(np.float32)
_ang_f = (2.0 * np.pi / NPIX) * np.outer(_j, _j)
_FC = np.cos(_ang_f).astype(np.float32)
_FS = np.sin(_ang_f).astype(np.float32)


def _fft_body(x_ref, ec_ref, es_ref, fc_ref, fs_ref, re_ref, im_ref):
    x = x_ref[0]
    wr = jnp.dot(x, ec_ref[...], preferred_element_type=jnp.float32)
    wi = -jnp.dot(x, es_ref[...], preferred_element_type=jnp.float32)
    fc = fc_ref[...]
    fs = fs_ref[...]
    zr = jnp.dot(fc, wr, preferred_element_type=jnp.float32) + \
         jnp.dot(fs, wi, preferred_element_type=jnp.float32)
    zi = jnp.dot(fc, wi, preferred_element_type=jnp.float32) - \
         jnp.dot(fs, wr, preferred_element_type=jnp.float32)
    re_ref[0] = _SCALE * zr
    im_ref[0] = _SCALE * zi


def _fft2(cube):
    spec_full = lambda shape: pl.BlockSpec(shape, lambda c: (0,) * len(shape))
    return pl.pallas_call(
        _fft_body,
        grid=(NCHAN,),
        in_specs=[
            pl.BlockSpec((1, NPIX, NPIX), lambda c: (c, 0, 0)),
            spec_full((NPIX, KPAD)),
            spec_full((NPIX, KPAD)),
            spec_full((NPIX, NPIX)),
            spec_full((NPIX, NPIX)),
        ],
        out_specs=[
            pl.BlockSpec((1, NPIX, KPAD), lambda c: (c, 0, 0)),
            pl.BlockSpec((1, NPIX, KPAD), lambda c: (c, 0, 0)),
        ],
        out_shape=[
            jax.ShapeDtypeStruct((NCHAN, NPIX, KPAD), jnp.float32),
            jax.ShapeDtypeStruct((NCHAN, NPIX, KPAD), jnp.float32),
        ],
    )(cube, _EC, _ES, _FC, _FS)


def kernel(cube, grid_mask):
    vis_re_p, vis_im_p = _fft2(cube)
    vis_re = vis_re_p[:, :, :KF].ravel()
    vis_im = vis_im_p[:, :, :KF].ravel()
    idx = jnp.nonzero(grid_mask.ravel(), size=NSEL)[0]
    return (vis_re[idx], vis_im[idx])
